# Initial kernel scaffold; baseline (speedup 1.0000x reference)
import jax
import jax.numpy as jnp
from jax.experimental import pallas as pl


def kernel(idx, tok_table, pos_table):
    raise NotImplementedError("write your pallas kernel here")


# speedup vs baseline: 1.3701x; 1.3701x over previous
import functools

import jax
import jax.numpy as jnp
from jax import lax
from jax.experimental import pallas as pl
from jax.experimental.pallas import tpu as pltpu
from jax.experimental.pallas import tpu_sc as plsc

D = 32
T = 200
NC = 2
NS = 16
NW = NC * NS

CHUNK = 800
SUB = 80
HALF = 16


@functools.partial(jax.jit, static_argnums=(3,))
def _emb(idx_flat, tok_table, pos_table, n_rows):
    per_w = n_rows // NW
    n_chunks = per_w // CHUNK
    seqs = CHUNK // T
    mesh = plsc.VectorSubcoreMesh(core_axis_name="c", subcore_axis_name="s")

    @functools.partial(
        pl.kernel,
        out_type=jax.ShapeDtypeStruct((n_rows, D), jnp.float32),
        mesh=mesh,
        scratch_types=[
            pltpu.VMEM((CHUNK,), jnp.int32),
            pltpu.VMEM((CHUNK, D), jnp.float32),
            pltpu.VMEM((T, D), jnp.float32),
            pltpu.SemaphoreType.DMA,
        ],
        compiler_params=pltpu.CompilerParams(use_tc_tiling_on_sc=False),
    )
    def body(idx_hbm, tok_hbm, pos_hbm, out_hbm, idx_v, rows_v, pos_v, sem):
        wid = lax.axis_index("s") * NC + lax.axis_index("c")
        base = wid * per_w
        pltpu.sync_copy(pos_hbm.at[pl.ds(0, T)], pos_v)

        def chunk_body(g, carry):
            off = base + g * CHUNK
            pltpu.sync_copy(idx_hbm.at[pl.ds(off, CHUNK)], idx_v)
            copies = [
                pltpu.async_copy(
                    tok_hbm.at[idx_v.at[pl.ds(j * SUB, SUB)]],
                    rows_v.at[pl.ds(j * SUB, SUB)],
                    sem,
                )
                for j in range(CHUNK // SUB)
            ]
            for c in copies:
                c.wait()

            def t_body(t, tc):
                p0 = pos_v[t, pl.ds(0, HALF)]
                p1 = pos_v[t, pl.ds(HALF, HALF)]
                for s in range(seqs):
                    r = s * T + t
                    rows_v[r, pl.ds(0, HALF)] = rows_v[r, pl.ds(0, HALF)] + p0
                    rows_v[r, pl.ds(HALF, HALF)] = (
                        rows_v[r, pl.ds(HALF, HALF)] + p1
                    )
                return tc

            lax.fori_loop(0, T, t_body, 0, unroll=2)
            pltpu.sync_copy(rows_v, out_hbm.at[pl.ds(off, CHUNK)])
            return carry

        lax.fori_loop(0, n_chunks, chunk_body, 0)

    return body(idx_flat, tok_table, pos_table)


def kernel(idx, tok_table, pos_table):
    b, t = idx.shape
    n_rows = b * t
    idx_flat = idx.reshape(n_rows).astype(jnp.int32)
    out = _emb(idx_flat, tok_table, pos_table, n_rows)
    return out.reshape(b, t, D)
